# baseline (speedup 1.0000x reference)
import functools

import jax
import jax.numpy as jnp
from jax import lax
from jax.experimental import pallas as pl
from jax.experimental.pallas import tpu as pltpu
from jax.experimental.pallas import tpu_sc as plsc

_NC = 2
_NS = 16
_NW = _NC * _NS


@functools.partial(jax.jit, static_argnames=("v", "d"))
def _sc_detile(table_t, v, d):
    nblk = (v + 127) // 128
    blkw = (nblk + _NW - 1) // _NW
    mesh = plsc.VectorSubcoreMesh(core_axis_name="c", subcore_axis_name="s")

    @functools.partial(
        pl.kernel,
        out_type=jax.ShapeDtypeStruct((v // 2 + 32, 128), jnp.float32),
        mesh=mesh,
        scratch_types=(
            [pltpu.VMEM((2, d, 128), jnp.float32),
             pltpu.VMEM((2, d, 128), jnp.float32)]
            + [pltpu.SemaphoreType.DMA] * 4
        ),
        compiler_params=pltpu.CompilerParams(
            needs_layout_passes=False, disable_bounds_checks=True),
    )
    def k1(tt_hbm, td_hbm, stg_v, trn_v, *sems):
        ss, ws = sems[:2], sems[2:]
        wid = lax.axis_index("s") * _NC + lax.axis_index("c")
        iota = lax.iota(jnp.int32, 16)
        qrows = [16 * q + iota for q in range(d // 16)]

        def stage(u, b):
            pltpu.async_copy(tt_hbm.at[:, pl.ds(u * 128, 128)], stg_v.at[b], ss[b])

        def unit(u, b):
            pltpu.make_async_copy(
                tt_hbm.at[:, pl.ds(0, 128)], stg_v.at[b], ss[b]).wait()

            @plsc.parallel_loop(0, 64, unroll=4)
            def _rows(rp):
                for par in range(2):
                    r = 2 * rp + par
                    rcol = jnp.full((16,), r, jnp.int32)
                    for q in range(d // 16):
                        vals = plsc.load_gather(stg_v.at[b], [qrows[q], rcol])
                        trn_v[b, rp, pl.ds(par * 64 + 16 * q, 16)] = vals

            pltpu.async_copy(trn_v.at[b], td_hbm.at[pl.ds(u * 64, 64)], ws[b])

        def wb_wait(b):
            pltpu.make_async_copy(
                trn_v.at[b], td_hbm.at[pl.ds(0, 64)], ws[b]).wait()

        u0 = wid * blkw

        @pl.when(u0 < nblk)
        def _():
            stage(u0, 0)

        @pl.loop(0, blkw, step=2)
        def _outer(i0):
            for b in range(2):
                i = i0 + b
                u = u0 + i

                @pl.when(u + 1 < jnp.minimum(u0 + blkw, nblk))
                def _():
                    stage(u + 1, 1 - b)

                @pl.when(jnp.logical_and(i < blkw, u < nblk))
                def _():
                    @pl.when(i >= 2)
                    def _():
                        wb_wait(b)

                    unit(u, b)

        for b in range(2):
            @pl.when(u0 + b < nblk)
            def _():
                wb_wait(b)

    return k1(table_t)


@functools.partial(jax.jit, static_argnames=("rows", "cols", "d", "ib"))
def _sc_gather(idxh, par, table_d, rows, cols, d, ib):
    nib = (rows // _NW) // ib
    nunit = cols * nib
    mesh = plsc.VectorSubcoreMesh(core_axis_name="c", subcore_axis_name="s")

    @functools.partial(
        pl.kernel,
        out_type=jax.ShapeDtypeStruct((cols, d, rows), jnp.float32),
        mesh=mesh,
        scratch_types=(
            [pltpu.VMEM((ib,), jnp.int32),
             pltpu.VMEM((ib,), jnp.int32),
             pltpu.VMEM((ib,), jnp.int32),
             pltpu.VMEM((ib,), jnp.int32),
             pltpu.VMEM((2, ib, 128), jnp.float32),
             pltpu.VMEM((2, d, ib), jnp.float32)]
            + [pltpu.SemaphoreType.DMA] * 6
        ),
        compiler_params=pltpu.CompilerParams(needs_layout_passes=False),
    )
    def k2(idx_hbm, par_hbm, td_hbm, out_hbm, i0_v, i1_v, p0_v, p1_v,
           rows_v, trn_v, *sems):
        gs, ps, ws = sems[:2], sems[2:4], sems[4:]
        idxs, pars = (i0_v, i1_v), (p0_v, p1_v)
        wid = lax.axis_index("s") * _NC + lax.axis_index("c")
        i_base = wid * (rows // _NW)
        iota = lax.iota(jnp.int32, 16)
        qrows = [16 * q + iota for q in range(d // 16)]

        def unit_ji(u):
            j = u // nib
            return j, i_base + (u - j * nib) * ib

        def fire(u, b):
            j, i0 = unit_ji(u)
            off = j * rows + i0
            pltpu.async_copy(par_hbm.at[pl.ds(off, ib)], pars[b], ps[b])
            pltpu.sync_copy(idx_hbm.at[pl.ds(off, ib)], idxs[b])
            pltpu.async_copy(td_hbm.at[idxs[b]], rows_v.at[b], gs[b])

        def complete(u, b):
            j, i0 = unit_ji(u)
            pltpu.make_async_copy(td_hbm.at[idxs[b]], rows_v.at[b], gs[b]).wait()
            pltpu.make_async_copy(par_hbm.at[pl.ds(0, ib)], pars[b], ps[b]).wait()

            @plsc.parallel_loop(0, ib // 16, unroll=4)
            def _iis(g):
                parv = pars[b][pl.ds(g * 16, 16)]
                for j16 in range(16):
                    ii = g * 16 + j16
                    off = parv[j16]
                    ccols = jnp.full((16,), ii, jnp.int32)
                    for q in range(d // 16):
                        vals = rows_v[b, ii, pl.ds(off + 16 * q, 16)]
                        plsc.store_scatter(trn_v.at[b], [qrows[q], ccols], vals)

            pltpu.async_copy(
                trn_v.at[b], out_hbm.at[j, :, pl.ds(i0, ib)], ws[b])

        def wb_wait(b):
            pltpu.make_async_copy(
                trn_v.at[b], out_hbm.at[0, :, pl.ds(0, ib)], ws[b]).wait()

        fire(jnp.int32(0), 0)

        @pl.loop(0, nunit, step=2)
        def _outer(u0):
            for b in range(2):
                u = u0 + b

                @pl.when(u + 1 < nunit)
                def _():
                    fire(u + 1, 1 - b)

                @pl.when(u >= 2)
                def _():
                    wb_wait(b)

                complete(u, b)

        for b in range(2):
            wb_wait(b)

    return k2(idxh, par, table_d)


def kernel(x, table):
    v, d = table.shape
    rows, cols = x.shape
    table_t = table.T
    table_d = _sc_detile(table_t, v, d)
    x_t = x.T.astype(jnp.int32)
    idxh = (x_t >> 1).reshape(rows * cols)
    par = ((x_t & 1) * d).reshape(rows * cols)
    out_k = _sc_gather(idxh, par, table_d, rows, cols, d, 256)
    return out_k.transpose(2, 0, 1)


# speedup vs baseline: 1.3411x; 1.3411x over previous
import functools

import jax
import jax.numpy as jnp
from jax import lax
from jax.experimental import pallas as pl
from jax.experimental.pallas import tpu as pltpu
from jax.experimental.pallas import tpu_sc as plsc

_NC = 2
_NS = 16
_NW = _NC * _NS


@functools.partial(jax.jit, static_argnames=("b_per_w", "chunk", "nbuf"))
def _sc_gather(idx, table, b_per_w, chunk, nbuf):
    B = idx.shape[0]
    D = table.shape[1]
    n = b_per_w // chunk
    assert n % nbuf == 0 and n >= nbuf
    mesh = plsc.VectorSubcoreMesh(core_axis_name="c", subcore_axis_name="s")

    @functools.partial(
        pl.kernel,
        out_type=jax.ShapeDtypeStruct((B, D), jnp.float32),
        mesh=mesh,
        scratch_types=(
            [pltpu.VMEM((nbuf, chunk), jnp.int32),
             pltpu.VMEM((nbuf, chunk, D), jnp.float32)]
            + [pltpu.SemaphoreType.DMA] * (2 * nbuf)
        ),
        compiler_params=pltpu.CompilerParams(use_tc_tiling_on_sc=False),
    )
    def k(idx_hbm, table_hbm, out_hbm, idx_v, rows_v, *sems):
        gs, ws = sems[:nbuf], sems[nbuf:]
        wid = lax.axis_index("s") * _NC + lax.axis_index("c")
        base = wid * b_per_w

        def fire_gather(i, b):
            off = base + i * chunk
            pltpu.sync_copy(idx_hbm.at[pl.ds(off, chunk)], idx_v.at[b])
            pltpu.async_copy(table_hbm.at[idx_v.at[b]], rows_v.at[b], gs[b])

        for b in range(nbuf - 1):
            fire_gather(jnp.int32(b), b)

        @pl.loop(0, n, step=nbuf)
        def _outer(i0):
            for b in range(nbuf):
                i = i0 + b
                sp = (b + nbuf - 1) % nbuf
                pre = i + nbuf - 1

                @pl.when(jnp.logical_and(pre < n, i >= 1))
                def _():
                    pltpu.make_async_copy(
                        rows_v.at[sp], out_hbm.at[pl.ds(base, chunk)], ws[sp]
                    ).wait()

                @pl.when(pre < n)
                def _():
                    fire_gather(pre, sp)

                pltpu.make_async_copy(
                    table_hbm.at[idx_v.at[b]], rows_v.at[b], gs[b]
                ).wait()
                off = base + i * chunk
                pltpu.async_copy(rows_v.at[b], out_hbm.at[pl.ds(off, chunk)], ws[b])

        for b in range(nbuf):
            pltpu.make_async_copy(
                rows_v.at[b], out_hbm.at[pl.ds(base, chunk)], ws[b]
            ).wait()

    return k(idx, table)


def kernel(x, table):
    B = x.size
    idx = x.reshape(B).astype(jnp.int32)
    b_per_w = B // _NW
    out = _sc_gather(idx, table, b_per_w, 800, 2)
    return out.reshape(*x.shape, table.shape[1])
